# baseline (device time: 400477 ns/iter reference)
import jax
import jax.numpy as jnp
from jax import lax
from jax.experimental import pallas as pl
from jax.experimental.pallas import tpu as pltpu

N_DEV = 8
SQ = 1024
SKV_SH = 1024
HQ = 8
DH = 128
D = HQ * DH
PW = D + 128
SCALE = 0.08838834764831843
BAND = 128
NGLOB = 32


def _partial_body(x_ref, wq_ref, k_ref, v_ref, p_ref):
    my = lax.axis_index("i")
    q = jnp.dot(x_ref[...], wq_ref[...], preferred_element_type=jnp.float32)
    qi = lax.broadcasted_iota(jnp.int32, (SQ, SKV_SH), 0)
    kj = lax.broadcasted_iota(jnp.int32, (SQ, SKV_SH), 1) + my * SKV_SH
    mask = (jnp.abs(qi - kj) <= BAND) | (kj < NGLOB) | (qi < NGLOB)
    p_ref[:, D:] = jnp.zeros((SQ, PW - D), jnp.float32)
    dens = []
    for h in range(HQ):
        qh = q[:, h * DH:(h + 1) * DH]
        kh = k_ref[:, h * DH:(h + 1) * DH]
        s = lax.dot_general(
            qh, kh, (((1,), (1,)), ((), ())),
            preferred_element_type=jnp.float32,
        ) * SCALE
        w = jnp.where(mask, jnp.exp(s), 0.0)
        p_ref[:, h * DH:(h + 1) * DH] = jnp.dot(
            w, v_ref[:, h * DH:(h + 1) * DH],
            preferred_element_type=jnp.float32,
        )
        dens.append(jnp.sum(w, axis=1, keepdims=True))
    p_ref[:, D:D + HQ] = jnp.concatenate(dens, axis=1)


def _allreduce_body(p_ref, s_ref, comm_ref, send_sems, recv_sems):
    my = lax.axis_index("i")
    left = jnp.mod(my - 1, N_DEV)
    right = jnp.mod(my + 1, N_DEV)

    barrier_sem = pltpu.get_barrier_semaphore()
    for nbr in (left, right):
        pl.semaphore_signal(
            barrier_sem, inc=1,
            device_id=(nbr,), device_id_type=pl.DeviceIdType.MESH,
        )
    pl.semaphore_wait(barrier_sem, 2)

    s_ref[...] = p_ref[...]
    comm_ref[0] = p_ref[...]

    for h in range(N_DEV - 1):
        send_slot = h % 2
        recv_slot = (h + 1) % 2
        rdma = pltpu.make_async_remote_copy(
            src_ref=comm_ref.at[send_slot],
            dst_ref=comm_ref.at[recv_slot],
            send_sem=send_sems.at[send_slot],
            recv_sem=recv_sems.at[recv_slot],
            device_id=(right,),
            device_id_type=pl.DeviceIdType.MESH,
        )
        rdma.start()
        rdma.wait()
        s_ref[...] += comm_ref[recv_slot]


def _final_body(s_ref, wo_ref, o_ref):
    parts = []
    for h in range(HQ):
        n = s_ref[:, h * DH:(h + 1) * DH]
        d = s_ref[:, D + h:D + h + 1]
        parts.append(n / d)
    ctx = jnp.concatenate(parts, axis=1)
    o_ref[...] = jnp.dot(ctx, wo_ref[...], preferred_element_type=jnp.float32)


def kernel(x, Wq, K_ext, V_ext, Wo):
    x2 = x[0]
    k2 = K_ext[0].reshape(SKV_SH, D)
    v2 = V_ext[0].reshape(SKV_SH, D)

    vmem = pl.BlockSpec(memory_space=pltpu.VMEM)

    partial = pl.pallas_call(
        _partial_body,
        out_shape=jax.ShapeDtypeStruct((SQ, PW), jnp.float32),
        in_specs=[vmem] * 4,
        out_specs=vmem,
    )(x2, Wq, k2, v2)

    summed = pl.pallas_call(
        _allreduce_body,
        out_shape=jax.ShapeDtypeStruct((SQ, PW), jnp.float32),
        in_specs=[vmem],
        out_specs=vmem,
        scratch_shapes=[
            pltpu.VMEM((2, SQ, PW), jnp.float32),
            pltpu.SemaphoreType.DMA((2,)),
            pltpu.SemaphoreType.DMA((2,)),
        ],
        compiler_params=pltpu.CompilerParams(collective_id=0),
    )(partial)

    out = pl.pallas_call(
        _final_body,
        out_shape=jax.ShapeDtypeStruct((SQ, D), jnp.float32),
        in_specs=[vmem] * 2,
        out_specs=vmem,
    )(summed, Wo)

    return out[None]


# device time: 107753 ns/iter; 3.7166x vs baseline; 3.7166x over previous
import functools

import jax
import jax.numpy as jnp
from jax import lax
from jax.experimental import pallas as pl
from jax.experimental.pallas import tpu as pltpu

N_DEV = 8
SQ = 1024
SKV_SH = 1024
HQ = 8
DH = 128
D = HQ * DH
PW = D + 128
BLK = SQ // N_DEV
SCALE = 0.08838834764831843
BAND = 128
NGLOB = 32

_MESH = pl.DeviceIdType.MESH


def _partial_body(x_ref, wq_ref, k_ref, v_ref, p_ref):
    my = lax.axis_index("i")
    q = jnp.dot(x_ref[...], wq_ref[...], preferred_element_type=jnp.float32)
    qi = lax.broadcasted_iota(jnp.int32, (SQ, SKV_SH), 0)
    kj = lax.broadcasted_iota(jnp.int32, (SQ, SKV_SH), 1) + my * SKV_SH
    mask = (jnp.abs(qi - kj) <= BAND) | (kj < NGLOB) | (qi < NGLOB)
    p_ref[:, D:] = jnp.zeros((SQ, PW - D), jnp.float32)
    dens = []
    for h in range(HQ):
        qh = q[:, h * DH:(h + 1) * DH]
        kh = k_ref[:, h * DH:(h + 1) * DH]
        s = lax.dot_general(
            qh, kh, (((1,), (1,)), ((), ())),
            preferred_element_type=jnp.float32,
        ) * SCALE
        w = jnp.where(mask, jnp.exp(s), 0.0)
        p_ref[:, h * DH:(h + 1) * DH] = jnp.dot(
            w, v_ref[:, h * DH:(h + 1) * DH],
            preferred_element_type=jnp.float32,
        )
        dens.append(jnp.sum(w, axis=1, keepdims=True))
    p_ref[:, D:D + HQ] = jnp.concatenate(dens, axis=1)


def _combine_ag_body(p_ref, wo_ref, o_ref,
                     c0_ref, cx_ref, bsum_ref,
                     c0_sems, cx_sems, s0_sems, s1_sems,
                     ag_ref, ag_send_sems, ag_recv_sems):
    my = lax.axis_index("i")
    right = jnp.mod(my + 1, N_DEV)

    barrier_sem = pltpu.get_barrier_semaphore()
    for p in range(N_DEV):
        @pl.when(my != p)
        def _(p=p):
            pl.semaphore_signal(
                barrier_sem, inc=1, device_id=(p,), device_id_type=_MESH)
    pl.semaphore_wait(barrier_sem, N_DEV - 1)

    @pl.when(my == 0)
    def _():
        for t in range(1, N_DEV):
            pltpu.make_async_remote_copy(
                src_ref=p_ref.at[pl.ds(t * BLK, BLK)],
                dst_ref=cx_ref.at[0],
                send_sem=s0_sems.at[t - 1],
                recv_sem=cx_sems.at[0],
                device_id=(t,), device_id_type=_MESH,
            ).start()

    for p in range(1, N_DEV):
        @pl.when(my == p)
        def _(p=p):
            pltpu.make_async_remote_copy(
                src_ref=p_ref.at[pl.ds(0, NGLOB)],
                dst_ref=c0_ref.at[p - 1],
                send_sem=s1_sems.at[0],
                recv_sem=c0_sems.at[p - 1],
                device_id=(0,), device_id_type=_MESH,
            ).start()

    @pl.when(my == 1)
    def _():
        pltpu.make_async_remote_copy(
            src_ref=p_ref.at[pl.ds(7 * BLK, BLK)],
            dst_ref=cx_ref.at[1],
            send_sem=s1_sems.at[1],
            recv_sem=cx_sems.at[1],
            device_id=(7,), device_id_type=_MESH,
        ).start()

    @pl.when(my == 0)
    def _():
        for k in range(N_DEV - 1):
            pltpu.make_async_remote_copy(
                src_ref=p_ref.at[pl.ds(0, NGLOB)],
                dst_ref=c0_ref.at[k],
                send_sem=s1_sems.at[0],
                recv_sem=c0_sems.at[k],
                device_id=(0,), device_id_type=_MESH,
            ).wait_recv()
        acc = c0_ref[0]
        for k in range(1, N_DEV - 1):
            acc = acc + c0_ref[k]
        bsum_ref[...] = p_ref[pl.ds(0, BLK)]
        bsum_ref[pl.ds(0, NGLOB)] = bsum_ref[pl.ds(0, NGLOB)] + acc

    @pl.when(my != 0)
    def _():
        pltpu.make_async_remote_copy(
            src_ref=p_ref.at[pl.ds(0, BLK)],
            dst_ref=cx_ref.at[0],
            send_sem=s1_sems.at[0],
            recv_sem=cx_sems.at[0],
            device_id=(0,), device_id_type=_MESH,
        ).wait_recv()
        bsum_ref[...] = cx_ref[0]

    @pl.when(my == 7)
    def _():
        pltpu.make_async_remote_copy(
            src_ref=p_ref.at[pl.ds(0, BLK)],
            dst_ref=cx_ref.at[1],
            send_sem=s1_sems.at[0],
            recv_sem=cx_sems.at[1],
            device_id=(0,), device_id_type=_MESH,
        ).wait_recv()
        bsum_ref[...] = bsum_ref[...] + cx_ref[1]

    @pl.when(my == 0)
    def _():
        for t in range(1, N_DEV):
            pltpu.make_async_remote_copy(
                src_ref=p_ref.at[pl.ds(t * BLK, BLK)],
                dst_ref=cx_ref.at[0],
                send_sem=s0_sems.at[t - 1],
                recv_sem=cx_sems.at[0],
                device_id=(t,), device_id_type=_MESH,
            ).wait_send()

    @pl.when(my != 0)
    def _():
        pltpu.make_async_remote_copy(
            src_ref=p_ref.at[pl.ds(0, NGLOB)],
            dst_ref=c0_ref.at[0],
            send_sem=s1_sems.at[0],
            recv_sem=c0_sems.at[0],
            device_id=(0,), device_id_type=_MESH,
        ).wait_send()

    @pl.when(my == 1)
    def _():
        pltpu.make_async_remote_copy(
            src_ref=p_ref.at[pl.ds(7 * BLK, BLK)],
            dst_ref=cx_ref.at[1],
            send_sem=s1_sems.at[1],
            recv_sem=cx_sems.at[1],
            device_id=(7,), device_id_type=_MESH,
        ).wait_send()

    parts = []
    for h in range(HQ):
        n = bsum_ref[:, h * DH:(h + 1) * DH]
        d = bsum_ref[:, D + h:D + h + 1]
        parts.append(n / d)
    ctx = jnp.concatenate(parts, axis=1)
    oblk = jnp.dot(ctx, wo_ref[...], preferred_element_type=jnp.float32)

    left = jnp.mod(my - 1, N_DEV)

    @functools.partial(pl.run_scoped, sem2=pltpu.SemaphoreType.REGULAR)
    def _(sem2):
        for nbr in (left, right):
            pl.semaphore_signal(
                sem2, inc=1, device_id=(nbr,), device_id_type=_MESH)
        pl.semaphore_wait(sem2, 2)

    o_ref[pl.ds(my * BLK, BLK)] = oblk
    ag_ref[0] = oblk
    for h in range(N_DEV - 1):
        ss = h % 2
        rs = (h + 1) % 2
        rdma = pltpu.make_async_remote_copy(
            src_ref=ag_ref.at[ss],
            dst_ref=ag_ref.at[rs],
            send_sem=ag_send_sems.at[ss],
            recv_sem=ag_recv_sems.at[rs],
            device_id=(right,), device_id_type=_MESH,
        )
        rdma.start()
        rdma.wait()
        origin = jnp.mod(my - h - 1, N_DEV)
        o_ref[pl.ds(origin * BLK, BLK)] = ag_ref[rs]


def kernel(x, Wq, K_ext, V_ext, Wo):
    x2 = x[0]
    k2 = K_ext[0].reshape(SKV_SH, D)
    v2 = V_ext[0].reshape(SKV_SH, D)

    vmem = pl.BlockSpec(memory_space=pltpu.VMEM)

    partial = pl.pallas_call(
        _partial_body,
        out_shape=jax.ShapeDtypeStruct((SQ, PW), jnp.float32),
        in_specs=[vmem] * 4,
        out_specs=vmem,
    )(x2, Wq, k2, v2)

    out = pl.pallas_call(
        _combine_ag_body,
        out_shape=jax.ShapeDtypeStruct((SQ, D), jnp.float32),
        in_specs=[vmem] * 2,
        out_specs=vmem,
        scratch_shapes=[
            pltpu.VMEM((N_DEV - 1, NGLOB, PW), jnp.float32),
            pltpu.VMEM((2, BLK, PW), jnp.float32),
            pltpu.VMEM((BLK, PW), jnp.float32),
            pltpu.SemaphoreType.DMA((N_DEV - 1,)),
            pltpu.SemaphoreType.DMA((2,)),
            pltpu.SemaphoreType.DMA((N_DEV - 1,)),
            pltpu.SemaphoreType.DMA((2,)),
            pltpu.VMEM((2, BLK, D), jnp.float32),
            pltpu.SemaphoreType.DMA((2,)),
            pltpu.SemaphoreType.DMA((2,)),
        ],
        compiler_params=pltpu.CompilerParams(collective_id=0),
    )(partial, Wo)

    return out[None]


# device time: 57916 ns/iter; 6.9148x vs baseline; 1.8605x over previous
import functools

import jax
import jax.numpy as jnp
from jax import lax
from jax.experimental import pallas as pl
from jax.experimental.pallas import tpu as pltpu

N_DEV = 8
SQ = 1024
SKV_SH = 1024
HQ = 8
DH = 128
D = HQ * DH
PW = D + 128
BLK = SQ // N_DEV
SCALE = 0.08838834764831843
BAND = 128
NGLOB = 32

_MESH = pl.DeviceIdType.MESH
_BF16 = jnp.bfloat16


def _partial_body(x_ref, wq_ref, k_ref, v_ref, p_ref):
    my = lax.axis_index("i")
    q = jnp.dot(x_ref[...], wq_ref[...], preferred_element_type=jnp.float32)
    qi = lax.broadcasted_iota(jnp.int32, (SQ, SKV_SH), 0)
    kj = lax.broadcasted_iota(jnp.int32, (SQ, SKV_SH), 1) + my * SKV_SH
    mask = (jnp.abs(qi - kj) <= BAND) | (kj < NGLOB) | (qi < NGLOB)
    p_ref[:, D:] = jnp.zeros((SQ, PW - D), _BF16)
    dens = []
    for h in range(HQ):
        qh = q[:, h * DH:(h + 1) * DH]
        kh = k_ref[:, h * DH:(h + 1) * DH]
        s = lax.dot_general(
            qh, kh, (((1,), (1,)), ((), ())),
            preferred_element_type=jnp.float32,
        ) * SCALE
        w = jnp.where(mask, jnp.exp(s), 0.0)
        p_ref[:, h * DH:(h + 1) * DH] = jnp.dot(
            w, v_ref[:, h * DH:(h + 1) * DH],
            preferred_element_type=jnp.float32,
        ).astype(_BF16)
        dens.append(jnp.sum(w, axis=1, keepdims=True))
    p_ref[:, D:D + HQ] = jnp.concatenate(dens, axis=1).astype(_BF16)


def _combine_ag_body(p_ref, wo_ref, o_ref,
                     c0_ref, cx_ref, bsum_ref, agsrc_ref, agdst_ref,
                     c0_sems, cx_sems, s0_sems, s1_sems,
                     ag_send_sems, ag_recv_sems):
    my = lax.axis_index("i")

    barrier_sem = pltpu.get_barrier_semaphore()
    for p in range(N_DEV):
        @pl.when(my != p)
        def _(p=p):
            pl.semaphore_signal(
                barrier_sem, inc=1, device_id=(p,), device_id_type=_MESH)
    pl.semaphore_wait(barrier_sem, N_DEV - 1)

    @pl.when(my == 0)
    def _():
        for t in range(1, N_DEV):
            pltpu.make_async_remote_copy(
                src_ref=p_ref.at[pl.ds(t * BLK, BLK)],
                dst_ref=cx_ref.at[0],
                send_sem=s0_sems.at[t - 1],
                recv_sem=cx_sems.at[0],
                device_id=(t,), device_id_type=_MESH,
            ).start()

    for p in range(1, N_DEV):
        @pl.when(my == p)
        def _(p=p):
            pltpu.make_async_remote_copy(
                src_ref=p_ref.at[pl.ds(0, NGLOB)],
                dst_ref=c0_ref.at[p - 1],
                send_sem=s1_sems.at[0],
                recv_sem=c0_sems.at[p - 1],
                device_id=(0,), device_id_type=_MESH,
            ).start()

    @pl.when(my == 1)
    def _():
        pltpu.make_async_remote_copy(
            src_ref=p_ref.at[pl.ds(7 * BLK, BLK)],
            dst_ref=cx_ref.at[1],
            send_sem=s1_sems.at[1],
            recv_sem=cx_sems.at[1],
            device_id=(7,), device_id_type=_MESH,
        ).start()

    @pl.when(my == 0)
    def _():
        for k in range(N_DEV - 1):
            pltpu.make_async_remote_copy(
                src_ref=p_ref.at[pl.ds(0, NGLOB)],
                dst_ref=c0_ref.at[k],
                send_sem=s1_sems.at[0],
                recv_sem=c0_sems.at[k],
                device_id=(0,), device_id_type=_MESH,
            ).wait_recv()
        acc = c0_ref[0].astype(jnp.float32)
        for k in range(1, N_DEV - 1):
            acc = acc + c0_ref[k].astype(jnp.float32)
        bsum_ref[...] = p_ref[pl.ds(0, BLK)].astype(jnp.float32)
        bsum_ref[pl.ds(0, NGLOB)] = bsum_ref[pl.ds(0, NGLOB)] + acc

    @pl.when(my != 0)
    def _():
        pltpu.make_async_remote_copy(
            src_ref=p_ref.at[pl.ds(0, BLK)],
            dst_ref=cx_ref.at[0],
            send_sem=s1_sems.at[0],
            recv_sem=cx_sems.at[0],
            device_id=(0,), device_id_type=_MESH,
        ).wait_recv()
        bsum_ref[...] = cx_ref[0].astype(jnp.float32)

    @pl.when(my == 7)
    def _():
        pltpu.make_async_remote_copy(
            src_ref=p_ref.at[pl.ds(0, BLK)],
            dst_ref=cx_ref.at[1],
            send_sem=s1_sems.at[0],
            recv_sem=cx_sems.at[1],
            device_id=(0,), device_id_type=_MESH,
        ).wait_recv()
        bsum_ref[...] = bsum_ref[...] + cx_ref[1].astype(jnp.float32)

    @pl.when(my == 0)
    def _():
        for t in range(1, N_DEV):
            pltpu.make_async_remote_copy(
                src_ref=p_ref.at[pl.ds(t * BLK, BLK)],
                dst_ref=cx_ref.at[0],
                send_sem=s0_sems.at[t - 1],
                recv_sem=cx_sems.at[0],
                device_id=(t,), device_id_type=_MESH,
            ).wait_send()

    @pl.when(my != 0)
    def _():
        pltpu.make_async_remote_copy(
            src_ref=p_ref.at[pl.ds(0, NGLOB)],
            dst_ref=c0_ref.at[0],
            send_sem=s1_sems.at[0],
            recv_sem=c0_sems.at[0],
            device_id=(0,), device_id_type=_MESH,
        ).wait_send()

    @pl.when(my == 1)
    def _():
        pltpu.make_async_remote_copy(
            src_ref=p_ref.at[pl.ds(7 * BLK, BLK)],
            dst_ref=cx_ref.at[1],
            send_sem=s1_sems.at[1],
            recv_sem=cx_sems.at[1],
            device_id=(7,), device_id_type=_MESH,
        ).wait_send()

    parts = []
    for h in range(HQ):
        n = bsum_ref[:, h * DH:(h + 1) * DH]
        d = bsum_ref[:, D + h:D + h + 1]
        parts.append(n / d)
    ctx = jnp.concatenate(parts, axis=1)
    oblk = jnp.dot(ctx, wo_ref[...], preferred_element_type=jnp.float32)

    o_ref[pl.ds(my * BLK, BLK)] = oblk
    agsrc_ref[...] = oblk.astype(_BF16)

    for t in range(N_DEV):
        @pl.when(my != t)
        def _(t=t):
            pltpu.make_async_remote_copy(
                src_ref=agsrc_ref,
                dst_ref=agdst_ref.at[my],
                send_sem=ag_send_sems.at[t],
                recv_sem=ag_recv_sems.at[my],
                device_id=(t,), device_id_type=_MESH,
            ).start()

    for k in range(N_DEV):
        @pl.when(my != k)
        def _(k=k):
            pltpu.make_async_remote_copy(
                src_ref=agsrc_ref,
                dst_ref=agdst_ref.at[k],
                send_sem=s1_sems.at[0],
                recv_sem=ag_recv_sems.at[k],
                device_id=(0,), device_id_type=_MESH,
            ).wait_recv()
            o_ref[k * BLK:(k + 1) * BLK] = agdst_ref[k].astype(jnp.float32)

    for t in range(N_DEV):
        @pl.when(my != t)
        def _(t=t):
            pltpu.make_async_remote_copy(
                src_ref=agsrc_ref,
                dst_ref=agdst_ref.at[0],
                send_sem=ag_send_sems.at[t],
                recv_sem=ag_recv_sems.at[0],
                device_id=(t,), device_id_type=_MESH,
            ).wait_send()

    @functools.partial(pl.run_scoped, sem2=pltpu.SemaphoreType.REGULAR)
    def _(sem2):
        for p in range(N_DEV):
            @pl.when(my != p)
            def _(p=p):
                pl.semaphore_signal(
                    sem2, inc=1, device_id=(p,), device_id_type=_MESH)
        pl.semaphore_wait(sem2, N_DEV - 1)


def kernel(x, Wq, K_ext, V_ext, Wo):
    x2 = x[0]
    k2 = K_ext[0].reshape(SKV_SH, D)
    v2 = V_ext[0].reshape(SKV_SH, D)

    vmem = pl.BlockSpec(memory_space=pltpu.VMEM)

    partial = pl.pallas_call(
        _partial_body,
        out_shape=jax.ShapeDtypeStruct((SQ, PW), _BF16),
        in_specs=[vmem] * 4,
        out_specs=vmem,
    )(x2, Wq, k2, v2)

    out = pl.pallas_call(
        _combine_ag_body,
        out_shape=jax.ShapeDtypeStruct((SQ, D), jnp.float32),
        in_specs=[vmem] * 2,
        out_specs=vmem,
        scratch_shapes=[
            pltpu.VMEM((N_DEV - 1, NGLOB, PW), _BF16),
            pltpu.VMEM((2, BLK, PW), _BF16),
            pltpu.VMEM((BLK, PW), jnp.float32),
            pltpu.VMEM((BLK, D), _BF16),
            pltpu.VMEM((N_DEV, BLK, D), _BF16),
            pltpu.SemaphoreType.DMA((N_DEV - 1,)),
            pltpu.SemaphoreType.DMA((2,)),
            pltpu.SemaphoreType.DMA((N_DEV - 1,)),
            pltpu.SemaphoreType.DMA((2,)),
            pltpu.SemaphoreType.DMA((N_DEV,)),
            pltpu.SemaphoreType.DMA((N_DEV,)),
        ],
        compiler_params=pltpu.CompilerParams(collective_id=0),
    )(partial, Wo)

    return out[None]


# device time: 54425 ns/iter; 7.3583x vs baseline; 1.0641x over previous
import functools

import jax
import jax.numpy as jnp
from jax import lax
from jax.experimental import pallas as pl
from jax.experimental.pallas import tpu as pltpu

N_DEV = 8
SQ = 1024
SKV_SH = 1024
HQ = 8
DH = 128
D = HQ * DH
PW = D + 128
BLK = SQ // N_DEV
KT = 128
NKT = SKV_SH // KT
SCALE = 0.08838834764831843
BAND = 128
NGLOB = 32

_MESH = pl.DeviceIdType.MESH
_BF16 = jnp.bfloat16
_F32 = jnp.float32


def _tiles_for_block(t):
    if t == 0:
        return list(range(NKT))
    return sorted({0, t - 1, t, t + 1} & set(range(NKT)))


def _block_partial_dev0(q, k_ref, v_ref, t):
    qi = t * BLK + lax.broadcasted_iota(jnp.int32, (BLK, KT), 0)
    cols = []
    dens = []
    for h in range(HQ):
        qh = q[t * BLK:(t + 1) * BLK, h * DH:(h + 1) * DH]
        num_h = jnp.zeros((BLK, DH), _F32)
        den_h = jnp.zeros((BLK, 1), _F32)
        for j in _tiles_for_block(t):
            kh = k_ref[j * KT:(j + 1) * KT, h * DH:(h + 1) * DH]
            s = lax.dot_general(
                qh, kh, (((1,), (1,)), ((), ())),
                preferred_element_type=_F32,
            ) * SCALE
            kj = j * KT + lax.broadcasted_iota(jnp.int32, (BLK, KT), 1)
            mask = (jnp.abs(qi - kj) <= BAND) | (kj < NGLOB) | (qi < NGLOB)
            w = jnp.where(mask, jnp.exp(s), 0.0)
            num_h = num_h + jnp.dot(
                w, v_ref[j * KT:(j + 1) * KT, h * DH:(h + 1) * DH],
                preferred_element_type=_F32)
            den_h = den_h + jnp.sum(w, axis=1, keepdims=True)
        cols.append(num_h)
        dens.append(den_h)
    den = jnp.concatenate(dens, axis=1)
    pad = jnp.zeros((BLK, PW - D - HQ), _F32)
    return jnp.concatenate(cols + [den, pad], axis=1).astype(_BF16)


def _combine_finalize(my, wo_ref, o_ref, bsum_ref, agsrc_ref, agdst_ref,
                      s1_sems, ag_send_sems, ag_recv_sems):
    parts = []
    for h in range(HQ):
        n = bsum_ref[:, h * DH:(h + 1) * DH]
        d = bsum_ref[:, D + h:D + h + 1]
        parts.append(n / d)
    ctx = jnp.concatenate(parts, axis=1)
    oblk = jnp.dot(ctx, wo_ref[...], preferred_element_type=_F32)

    o_ref[pl.ds(my * BLK, BLK)] = oblk
    agsrc_ref[...] = oblk.astype(_BF16)

    for t in range(N_DEV):
        @pl.when(my != t)
        def _(t=t):
            pltpu.make_async_remote_copy(
                src_ref=agsrc_ref,
                dst_ref=agdst_ref.at[my],
                send_sem=ag_send_sems.at[t],
                recv_sem=ag_recv_sems.at[my],
                device_id=(t,), device_id_type=_MESH,
            ).start()

    for k in range(N_DEV):
        @pl.when(my != k)
        def _(k=k):
            pltpu.make_async_remote_copy(
                src_ref=agsrc_ref,
                dst_ref=agdst_ref.at[k],
                send_sem=s1_sems.at[0],
                recv_sem=ag_recv_sems.at[k],
                device_id=(0,), device_id_type=_MESH,
            ).wait_recv()
            o_ref[k * BLK:(k + 1) * BLK] = agdst_ref[k].astype(_F32)

    for t in range(N_DEV):
        @pl.when(my != t)
        def _(t=t):
            pltpu.make_async_remote_copy(
                src_ref=agsrc_ref,
                dst_ref=agdst_ref.at[0],
                send_sem=ag_send_sems.at[t],
                recv_sem=ag_recv_sems.at[0],
                device_id=(t,), device_id_type=_MESH,
            ).wait_send()


def _fused_body(x_ref, wq_ref, k_ref, v_ref, wo_ref, o_ref,
                pstage_ref, p32_ref, px_ref,
                c0_ref, cx_ref, bsum_ref, agsrc_ref, agdst_ref,
                c0_sems, cx_sems, s0_sems, s1_sems,
                ag_send_sems, ag_recv_sems):
    my = lax.axis_index("i")

    barrier_sem = pltpu.get_barrier_semaphore()
    for p in range(N_DEV):
        @pl.when(my != p)
        def _(p=p):
            pl.semaphore_signal(
                barrier_sem, inc=1, device_id=(p,), device_id_type=_MESH)
    pl.semaphore_wait(barrier_sem, N_DEV - 1)

    @pl.when(my == 0)
    def _():
        q = jnp.dot(x_ref[...], wq_ref[...], preferred_element_type=_F32)
        for t in range(1, N_DEV):
            pstage_ref[t - 1] = _block_partial_dev0(q, k_ref, v_ref, t)
            pltpu.make_async_remote_copy(
                src_ref=pstage_ref.at[t - 1],
                dst_ref=cx_ref.at[0],
                send_sem=s0_sems.at[t - 1],
                recv_sem=cx_sems.at[0],
                device_id=(t,), device_id_type=_MESH,
            ).start()
        bsum_ref[...] = _block_partial_dev0(q, k_ref, v_ref, 0).astype(_F32)

    @pl.when(my != 0)
    def _():
        qg = jnp.dot(x_ref[pl.ds(0, NGLOB)], wq_ref[...],
                     preferred_element_type=_F32)
        cols = []
        dens = []
        for h in range(HQ):
            qh = qg[:, h * DH:(h + 1) * DH]
            kh = k_ref[:, h * DH:(h + 1) * DH]
            s = lax.dot_general(
                qh, kh, (((1,), (1,)), ((), ())),
                preferred_element_type=_F32,
            ) * SCALE
            w = jnp.exp(s)
            cols.append(jnp.dot(w, v_ref[:, h * DH:(h + 1) * DH],
                                preferred_element_type=_F32))
            dens.append(jnp.sum(w, axis=1, keepdims=True))
        den = jnp.concatenate(dens, axis=1)
        pad = jnp.zeros((NGLOB, PW - D - HQ), _F32)
        p32_ref[...] = jnp.concatenate(cols + [den, pad], axis=1).astype(_BF16)

    for p in range(1, N_DEV):
        @pl.when(my == p)
        def _(p=p):
            pltpu.make_async_remote_copy(
                src_ref=p32_ref,
                dst_ref=c0_ref.at[p - 1],
                send_sem=s1_sems.at[0],
                recv_sem=c0_sems.at[p - 1],
                device_id=(0,), device_id_type=_MESH,
            ).start()

    @pl.when(my == 1)
    def _():
        q9 = jnp.dot(x_ref[pl.ds(7 * BLK, BLK)], wq_ref[...],
                     preferred_element_type=_F32)
        qi = 7 * BLK + lax.broadcasted_iota(jnp.int32, (BLK, KT), 0)
        kj = SKV_SH + lax.broadcasted_iota(jnp.int32, (BLK, KT), 1)
        mask = jnp.abs(qi - kj) <= BAND
        cols = []
        dens = []
        for h in range(HQ):
            qh = q9[:, h * DH:(h + 1) * DH]
            kh = k_ref[0:KT, h * DH:(h + 1) * DH]
            s = lax.dot_general(
                qh, kh, (((1,), (1,)), ((), ())),
                preferred_element_type=_F32,
            ) * SCALE
            w = jnp.where(mask, jnp.exp(s), 0.0)
            cols.append(jnp.dot(w, v_ref[0:KT, h * DH:(h + 1) * DH],
                                preferred_element_type=_F32))
            dens.append(jnp.sum(w, axis=1, keepdims=True))
        den = jnp.concatenate(dens, axis=1)
        pad = jnp.zeros((BLK, PW - D - HQ), _F32)
        px_ref[...] = jnp.concatenate(cols + [den, pad], axis=1).astype(_BF16)
        pltpu.make_async_remote_copy(
            src_ref=px_ref,
            dst_ref=cx_ref.at[1],
            send_sem=s1_sems.at[1],
            recv_sem=cx_sems.at[1],
            device_id=(7,), device_id_type=_MESH,
        ).start()

    @pl.when(my == 0)
    def _():
        for k in range(N_DEV - 1):
            pltpu.make_async_remote_copy(
                src_ref=p32_ref,
                dst_ref=c0_ref.at[k],
                send_sem=s1_sems.at[0],
                recv_sem=c0_sems.at[k],
                device_id=(0,), device_id_type=_MESH,
            ).wait_recv()
        acc = c0_ref[0].astype(_F32)
        for k in range(1, N_DEV - 1):
            acc = acc + c0_ref[k].astype(_F32)
        bsum_ref[pl.ds(0, NGLOB)] = bsum_ref[pl.ds(0, NGLOB)] + acc

    @pl.when(my != 0)
    def _():
        pltpu.make_async_remote_copy(
            src_ref=px_ref,
            dst_ref=cx_ref.at[0],
            send_sem=s1_sems.at[0],
            recv_sem=cx_sems.at[0],
            device_id=(0,), device_id_type=_MESH,
        ).wait_recv()
        bsum_ref[...] = cx_ref[0].astype(_F32)

    @pl.when(my == 7)
    def _():
        pltpu.make_async_remote_copy(
            src_ref=px_ref,
            dst_ref=cx_ref.at[1],
            send_sem=s1_sems.at[0],
            recv_sem=cx_sems.at[1],
            device_id=(0,), device_id_type=_MESH,
        ).wait_recv()
        bsum_ref[...] = bsum_ref[...] + cx_ref[1].astype(_F32)

    @pl.when(my == 0)
    def _():
        for t in range(1, N_DEV):
            pltpu.make_async_remote_copy(
                src_ref=pstage_ref.at[t - 1],
                dst_ref=cx_ref.at[0],
                send_sem=s0_sems.at[t - 1],
                recv_sem=cx_sems.at[0],
                device_id=(t,), device_id_type=_MESH,
            ).wait_send()

    @pl.when(my != 0)
    def _():
        pltpu.make_async_remote_copy(
            src_ref=p32_ref,
            dst_ref=c0_ref.at[0],
            send_sem=s1_sems.at[0],
            recv_sem=c0_sems.at[0],
            device_id=(0,), device_id_type=_MESH,
        ).wait_send()

    @pl.when(my == 1)
    def _():
        pltpu.make_async_remote_copy(
            src_ref=px_ref,
            dst_ref=cx_ref.at[1],
            send_sem=s1_sems.at[1],
            recv_sem=cx_sems.at[1],
            device_id=(7,), device_id_type=_MESH,
        ).wait_send()

    _combine_finalize(my, wo_ref, o_ref, bsum_ref, agsrc_ref, agdst_ref,
                      s1_sems, ag_send_sems, ag_recv_sems)

    @functools.partial(pl.run_scoped, sem2=pltpu.SemaphoreType.REGULAR)
    def _(sem2):
        for p in range(N_DEV):
            @pl.when(my != p)
            def _(p=p):
                pl.semaphore_signal(
                    sem2, inc=1, device_id=(p,), device_id_type=_MESH)
        pl.semaphore_wait(sem2, N_DEV - 1)


def kernel(x, Wq, K_ext, V_ext, Wo):
    x2 = x[0]
    k2 = K_ext[0].reshape(SKV_SH, D)
    v2 = V_ext[0].reshape(SKV_SH, D)

    vmem = pl.BlockSpec(memory_space=pltpu.VMEM)

    out = pl.pallas_call(
        _fused_body,
        out_shape=jax.ShapeDtypeStruct((SQ, D), jnp.float32),
        in_specs=[vmem] * 5,
        out_specs=vmem,
        scratch_shapes=[
            pltpu.VMEM((N_DEV - 1, BLK, PW), _BF16),
            pltpu.VMEM((NGLOB, PW), _BF16),
            pltpu.VMEM((BLK, PW), _BF16),
            pltpu.VMEM((N_DEV - 1, NGLOB, PW), _BF16),
            pltpu.VMEM((2, BLK, PW), _BF16),
            pltpu.VMEM((BLK, PW), jnp.float32),
            pltpu.VMEM((BLK, D), _BF16),
            pltpu.VMEM((N_DEV, BLK, D), _BF16),
            pltpu.SemaphoreType.DMA((N_DEV - 1,)),
            pltpu.SemaphoreType.DMA((2,)),
            pltpu.SemaphoreType.DMA((N_DEV - 1,)),
            pltpu.SemaphoreType.DMA((2,)),
            pltpu.SemaphoreType.DMA((N_DEV,)),
            pltpu.SemaphoreType.DMA((N_DEV,)),
        ],
        compiler_params=pltpu.CompilerParams(collective_id=0),
    )(x2, Wq, k2, v2, Wo)

    return out[None]
